# baseline (speedup 1.0000x reference)
import jax
import jax.numpy as jnp
from jax.experimental import pallas as pl
from jax.experimental.pallas import tpu as pltpu

B = 524288
D = 128
A = 21
AP = 24
TB = 16384


def _folds(v, op):
    return op(op(v[0:8], v[8:16]), v[16:24])


def _sum_a(v):
    return jnp.sum(_folds(v, jnp.add), axis=0, keepdims=True)


def _one(xt, act, wt, bp, tlo, thi):
    l = jax.lax.dot_general(
        wt, xt, dimension_numbers=(((1,), (1,)), ((), ())),
        preferred_element_type=jnp.float32) + bp
    ex = jnp.exp(l)
    s = _sum_a(ex)
    rs = 1.0 / s
    u = _sum_a(ex * l)
    ent = (jnp.log(s) - u * rs) * (1.0 / jnp.log(float(A)))
    two_a = act + act
    hot_ex = jnp.where(two_a >= tlo, ex, 0.0) - jnp.where(
        two_a >= thi, ex, 0.0)
    sel = _sum_a(hot_ex) * rs
    return sel, ent


def _fused_kernel(x1_ref, x2_ref, a_ref, wt_ref, b_ref, tlo_ref, thi_ref,
                  sel_ref, ent_ref):
    wt, bp = wt_ref[...], b_ref[...]
    tlo, thi = tlo_ref[...], thi_ref[...]
    sel1, ent1 = _one(x1_ref[...], a_ref[0], wt, bp, tlo, thi)
    sel_ref[0] = sel1
    ent_ref[0] = ent1
    sel2, ent2 = _one(x2_ref[...], a_ref[1], wt, bp, tlo, thi)
    sel_ref[1] = sel2
    ent_ref[1] = ent2


def _thresholds(action_values):
    lo, hi = action_values[:-1], action_values[1:]
    s = lo + hi
    e = hi - (s - lo)
    tau = jnp.where(e >= 0, jnp.nextafter(s, jnp.inf), s)
    thi = jnp.full((AP, 1), jnp.inf, dtype=jnp.float32).at[: A - 1, 0].set(tau)
    tlo = jnp.full((AP, 1), jnp.inf, dtype=jnp.float32)
    tlo = tlo.at[1:A, 0].set(tau).at[0, 0].set(-jnp.inf)
    return tlo, thi


def kernel(x, actions, W, b, action_values):
    nb = B // TB
    wt = jnp.zeros((AP, D), dtype=jnp.float32).at[:A, :].set(W)
    bp = jnp.full((AP, 1), -1e30, dtype=jnp.float32).at[:A, 0].set(b)
    tlo, thi = _thresholds(action_values)
    act3 = actions.reshape(nb, 1, TB)

    sel, ent = pl.pallas_call(
        _fused_kernel,
        grid=(nb // 2,),
        in_specs=[
            pl.BlockSpec((TB, D), lambda i: (2 * i, 0)),
            pl.BlockSpec((TB, D), lambda i: (2 * i + 1, 0)),
            pl.BlockSpec((2, 1, TB), lambda i: (i, 0, 0)),
            pl.BlockSpec((AP, D), lambda i: (0, 0)),
            pl.BlockSpec((AP, 1), lambda i: (0, 0)),
            pl.BlockSpec((AP, 1), lambda i: (0, 0)),
            pl.BlockSpec((AP, 1), lambda i: (0, 0)),
        ],
        out_specs=[
            pl.BlockSpec((2, 1, TB), lambda i: (i, 0, 0)),
            pl.BlockSpec((2, 1, TB), lambda i: (i, 0, 0)),
        ],
        out_shape=[
            jax.ShapeDtypeStruct((nb, 1, TB), jnp.float32),
            jax.ShapeDtypeStruct((nb, 1, TB), jnp.float32),
        ],
        compiler_params=pltpu.CompilerParams(
            dimension_semantics=("parallel",),
        ),
    )(x, x, act3, wt, bp, tlo, thi)
    return sel.reshape(B), ent.reshape(B)


# speedup vs baseline: 1.0502x; 1.0502x over previous
import jax
import jax.numpy as jnp
from jax.experimental import pallas as pl
from jax.experimental.pallas import tpu as pltpu

B = 524288
D = 128
A = 21
AP = 24
TB = 32768


def _folds(v, op):
    return op(op(v[0:8], v[8:16]), v[16:24])


def _sum_a(v):
    return jnp.sum(_folds(v, jnp.add), axis=0, keepdims=True)


def _fused_kernel(x_ref, a_ref, wt_ref, b_ref, tlo_ref, thi_ref,
                  sel_ref, ent_ref):
    xt = x_ref[...]
    l = jax.lax.dot_general(
        wt_ref[...], xt, dimension_numbers=(((1,), (1,)), ((), ())),
        preferred_element_type=jnp.float32) + b_ref[...]
    ex = jnp.exp(l)
    s = _sum_a(ex)
    rs = 1.0 / s
    u = _sum_a(ex * l)
    ent = (jnp.log(s) - u * rs) * (1.0 / jnp.log(float(A)))

    two_a = a_ref[0] + a_ref[0]
    hot_ex = jnp.where(two_a >= tlo_ref[...], ex, 0.0) - jnp.where(
        two_a >= thi_ref[...], ex, 0.0)
    sel = _sum_a(hot_ex) * rs

    sel_ref[0] = sel
    ent_ref[0] = ent


def _thresholds(action_values):
    lo, hi = action_values[:-1], action_values[1:]
    s = lo + hi
    e = hi - (s - lo)
    tau = jnp.where(e >= 0, jnp.nextafter(s, jnp.inf), s)
    thi = jnp.full((AP, 1), jnp.inf, dtype=jnp.float32).at[: A - 1, 0].set(tau)
    tlo = jnp.full((AP, 1), jnp.inf, dtype=jnp.float32)
    tlo = tlo.at[1:A, 0].set(tau).at[0, 0].set(-jnp.inf)
    return tlo, thi


def kernel(x, actions, W, b, action_values):
    nb = B // TB
    wt = jnp.zeros((AP, D), dtype=jnp.float32).at[:A, :].set(W)
    bp = jnp.full((AP, 1), -1e30, dtype=jnp.float32).at[:A, 0].set(b)
    tlo, thi = _thresholds(action_values)
    act3 = actions.reshape(nb, 1, TB)

    sel, ent = pl.pallas_call(
        _fused_kernel,
        grid=(nb,),
        in_specs=[
            pl.BlockSpec((TB, D), lambda i: (i, 0)),
            pl.BlockSpec((1, 1, TB), lambda i: (i, 0, 0)),
            pl.BlockSpec((AP, D), lambda i: (0, 0)),
            pl.BlockSpec((AP, 1), lambda i: (0, 0)),
            pl.BlockSpec((AP, 1), lambda i: (0, 0)),
            pl.BlockSpec((AP, 1), lambda i: (0, 0)),
        ],
        out_specs=[
            pl.BlockSpec((1, 1, TB), lambda i: (i, 0, 0)),
            pl.BlockSpec((1, 1, TB), lambda i: (i, 0, 0)),
        ],
        out_shape=[
            jax.ShapeDtypeStruct((nb, 1, TB), jnp.float32),
            jax.ShapeDtypeStruct((nb, 1, TB), jnp.float32),
        ],
        compiler_params=pltpu.CompilerParams(
            dimension_semantics=("parallel",),
        ),
    )(x, act3, wt, bp, tlo, thi)
    return sel.reshape(B), ent.reshape(B)
